# baseline (speedup 1.0000x reference)
import functools

import jax
import jax.numpy as jnp
from jax import lax
from jax.experimental import pallas as pl
from jax.experimental.pallas import tpu as pltpu, tpu_sc as plsc

_PI = 3.1415926


def _nodestats_body(n, bn, f_ref, p_ref, of_ref, og_ref, op_ref, os_ref):
    i = pl.program_id(0)
    f = f_ref[...]
    p = p_ref[...]
    t = f.shape[1]
    rows = i * bn + lax.broadcasted_iota(jnp.int32, (bn,), 0)
    m = rows < n
    zero = jnp.zeros((bn,), jnp.float32)
    of_ref[...] = jnp.where(m, jnp.sum(f, axis=1), zero)
    og_ref[...] = jnp.where(m, f[:, t - 1] - f[:, 0], zero)
    op_ref[...] = jnp.where(m, jnp.sum(p, axis=1), zero)
    os_ref[...] = jnp.where(m, jnp.sum(f * f, axis=1), zero)


def _node_stats(flowrate, pressure, n_pad):
    n, t = flowrate.shape
    bn = 2048
    grid = (n_pad // bn,)
    return pl.pallas_call(
        functools.partial(_nodestats_body, n, bn),
        grid=grid,
        in_specs=[
            pl.BlockSpec((bn, t), lambda i: (i, 0)),
            pl.BlockSpec((bn, t), lambda i: (i, 0)),
        ],
        out_specs=[pl.BlockSpec((bn,), lambda i: (i,))] * 4,
        out_shape=[jax.ShapeDtypeStruct((n_pad,), jnp.float32)] * 4,
    )(flowrate, pressure)


def _edge_body(n_pad, epw, e0, has_prior, *refs):
    if has_prior:
        (prior_h, ft_h, gt_h, pt_h, src_h, dst_h, ln_h, dm_h,
         part_out, laste_out,
         ft_v, gt_v, pt_v, lt_v, sb, db, lb, mb, accb, sem) = refs
    else:
        (ft_h, gt_h, pt_h, src_h, dst_h, ln_h, dm_h,
         part_out, laste_out,
         ft_v, gt_v, pt_v, lt_v, sb, db, lb, mb, accb, sem) = refs
    c = lax.axis_index("c")
    s = lax.axis_index("s")
    nc = lax.axis_size("c")
    wid = s * nc + c
    aoff = wid * epw
    ebase = e0 + aoff
    lane = lax.iota(jnp.int32, 16)

    cps = [
        pltpu.async_copy(ft_h, ft_v, sem),
        pltpu.async_copy(gt_h, gt_v, sem),
        pltpu.async_copy(pt_h, pt_v, sem),
        pltpu.async_copy(src_h.at[pl.ds(aoff, epw)], sb, sem),
        pltpu.async_copy(dst_h.at[pl.ds(aoff, epw)], db, sem),
        pltpu.async_copy(ln_h.at[pl.ds(aoff, epw)], lb, sem),
        pltpu.async_copy(dm_h.at[pl.ds(aoff, epw)], mb, sem),
    ]
    if has_prior:
        cps.append(pltpu.async_copy(prior_h.at[wid], lt_v, sem))
    for cp in cps:
        cp.wait()

    if not has_prior:
        def init_body(j, _):
            lt_v[pl.ds(j * 16, 16)] = jnp.full((16,), -1, jnp.int32)
            return 0
        lax.fori_loop(0, n_pad // 16, init_body, 0)

    unroll = 2

    def group_body(it, carry):
        av, au, ap = carry
        for u in range(unroll):
            g = it * unroll + u
            sl = pl.ds(g * 16, 16)
            si = sb[sl]
            di = db[sl]
            ln = lb[sl]
            dm = mb[sl]
            fs = plsc.load_gather(ft_v, [si])
            fd = plsc.load_gather(ft_v, [di])
            gs = plsc.load_gather(gt_v, [si])
            gd = plsc.load_gather(gt_v, [di])
            ps = plsc.load_gather(pt_v, [si])
            pd = plsc.load_gather(pt_v, [di])
            d2 = dm * dm
            d4 = d2 * d2
            av = av + ln / d4 * (fs - fd)
            au = au + ln / d2 * (gs - gd)
            ap = ap + (ps - pd)
            e = (ebase + g * 16) + lane
            plsc.store_scatter(lt_v, [di], e)
        return av, au, ap

    zero16 = jnp.zeros((16,), jnp.float32)
    av, au, ap = lax.fori_loop(0, epw // 16 // unroll, group_body,
                               (zero16, zero16, zero16))

    accb[0, :] = av
    accb[1, :] = au
    accb[2, :] = ap
    accb[3, :] = zero16
    pltpu.sync_copy(accb, part_out.at[wid])
    pltpu.sync_copy(lt_v, laste_out.at[wid])


def _merge_body(n, nsl, nw, e2, laste_hbm, sq_h, dma_h, dmb_h, kin_out,
                l_v, ml_v, wia_v, wib_v, da_v, db_v, sq_v, ob, sem):
    c = lax.axis_index("c")
    s = lax.axis_index("s")
    nc = lax.axis_size("c")
    wid = s * nc + c
    nbase = wid * nsl
    lane = lax.iota(jnp.int32, 16)

    pltpu.sync_copy(laste_hbm.at[:, pl.ds(nbase, nsl)], l_v)
    pltpu.sync_copy(sq_h.at[pl.ds(nbase, nsl)], sq_v)

    def mbody(j, _):
        sl = pl.ds(j * 16, 16)
        ml = l_v[0, sl]
        for ti in range(1, nw):
            ml = jnp.maximum(ml, l_v[ti, sl])
        ml_v[sl] = ml
        mlc = jnp.maximum(ml, 0)
        wia_v[sl] = jnp.minimum(mlc, e2 - 1)
        wib_v[sl] = jnp.minimum(jnp.maximum(ml - e2, 0), e2 - 1)
        return 0
    lax.fori_loop(0, nsl // 16, mbody, 0)

    pick0 = (lane == 0) & (wid == 0)
    v0 = ml_v[pl.ds(0, 16)]
    ml_v[pl.ds(0, 16)] = jnp.where(pick0, v0[1], v0)
    wa = wia_v[pl.ds(0, 16)]
    wia_v[pl.ds(0, 16)] = jnp.where(pick0, wa[1], wa)
    wb = wib_v[pl.ds(0, 16)]
    wib_v[pl.ds(0, 16)] = jnp.where(pick0, wb[1], wb)

    cpa = pltpu.async_copy(dma_h.at[wia_v], da_v, sem)
    cpa.wait()
    cpb = pltpu.async_copy(dmb_h.at[wib_v], db_v, sem)
    cpb.wait()

    def kbody(j, acc):
        sl = pl.ds(j * 16, 16)
        ml = ml_v[sl]
        one = jnp.full((16,), 1.0, jnp.float32)
        d = jnp.where(ml < e2, da_v[sl], db_v[sl])
        d = jnp.where(ml < 0, one, d)
        sq = sq_v[sl]
        d2 = d * d
        return acc + sq / (d2 * d2)
    acc = lax.fori_loop(0, nsl // 16, kbody, jnp.zeros((16,), jnp.float32))
    ob[...] = acc
    pltpu.sync_copy(ob, kin_out.at[wid])


def kernel(flowrate, pressure, edge_attr, edge_index, rho, vis, total_time):
    n, t = flowrate.shape
    e = edge_index.shape[1]

    info = plsc.get_sparse_core_info()
    nc, ns = info.num_cores, info.num_subcores
    nw = nc * ns
    n_pad = ((n + nw * 16 - 1) // (nw * 16)) * (nw * 16)
    e2 = e // 2
    epw = e2 // nw
    nsl = n_pad // nw

    src_a = edge_index[0, :e2]
    dst_a = edge_index[1, :e2]
    ln_a = edge_attr[:e2, 0]
    dm_a = edge_attr[:e2, 1]
    src_b = edge_index[0, e2:]
    dst_b = edge_index[1, e2:]
    ln_b = edge_attr[e2:, 0]
    dm_b = edge_attr[e2:, 1]

    ft, gt, pt, sq = _node_stats(flowrate, pressure, n_pad)

    mesh = plsc.VectorSubcoreMesh(core_axis_name="c", subcore_axis_name="s")
    sc_params = pltpu.CompilerParams(needs_layout_passes=False,
                                     use_tc_tiling_on_sc=False)
    edge_out = (
        jax.ShapeDtypeStruct((nw, 4, 16), jnp.float32),
        jax.ShapeDtypeStruct((nw, n_pad), jnp.int32),
    )
    edge_scratch = [
        pltpu.VMEM((n_pad,), jnp.float32),
        pltpu.VMEM((n_pad,), jnp.float32),
        pltpu.VMEM((n_pad,), jnp.float32),
        pltpu.VMEM((n_pad,), jnp.int32),
        pltpu.VMEM((epw,), jnp.int32),
        pltpu.VMEM((epw,), jnp.int32),
        pltpu.VMEM((epw,), jnp.float32),
        pltpu.VMEM((epw,), jnp.float32),
        pltpu.VMEM((4, 16), jnp.float32),
        pltpu.SemaphoreType.DMA,
    ]

    edge_ka = functools.partial(
        pl.kernel, out_type=edge_out, mesh=mesh,
        scratch_types=edge_scratch, compiler_params=sc_params,
    )(functools.partial(_edge_body, n_pad, epw, 0, False))
    part_a, laste_a = edge_ka(ft, gt, pt, src_a, dst_a, ln_a, dm_a)

    edge_kb = functools.partial(
        pl.kernel, out_type=edge_out, mesh=mesh,
        scratch_types=edge_scratch, compiler_params=sc_params,
    )(functools.partial(_edge_body, n_pad, epw, e2, True))
    part_b, laste_b = edge_kb(laste_a, ft, gt, pt, src_b, dst_b, ln_b, dm_b)

    merge_k = functools.partial(
        pl.kernel,
        out_type=jax.ShapeDtypeStruct((nw, 16), jnp.float32),
        mesh=mesh,
        scratch_types=[
            pltpu.VMEM((nw, nsl), jnp.int32),
            pltpu.VMEM((nsl,), jnp.int32),
            pltpu.VMEM((nsl,), jnp.int32),
            pltpu.VMEM((nsl,), jnp.int32),
            pltpu.VMEM((nsl,), jnp.float32),
            pltpu.VMEM((nsl,), jnp.float32),
            pltpu.VMEM((nsl,), jnp.float32),
            pltpu.VMEM((16,), jnp.float32),
            pltpu.SemaphoreType.DMA,
        ],
        compiler_params=sc_params,
    )(functools.partial(_merge_body, n, nsl, nw, e2))
    kin_part = merge_k(laste_b, sq, dm_a, dm_b)

    s_vis = jnp.sum(part_a[:, 0, :]) + jnp.sum(part_b[:, 0, :])
    s_uns = jnp.sum(part_a[:, 1, :]) + jnp.sum(part_b[:, 1, :])
    s_p = jnp.sum(part_a[:, 2, :]) + jnp.sum(part_b[:, 2, :])
    s_kin = jnp.sum(kin_part)

    rho0 = rho[0]
    loss = (16.0 * rho0 / (_PI * _PI)) * s_kin / (n * t)
    loss = loss + (128.0 * vis[0] / _PI) * s_vis / (e * t)
    loss = loss + (4.0 * rho0 / (_PI * total_time[0])) * s_uns / e
    loss = loss + s_p / (e * t)
    return loss


# speedup vs baseline: 1.9129x; 1.9129x over previous
import functools

import jax
import jax.numpy as jnp
from jax import lax
from jax.experimental import pallas as pl
from jax.experimental.pallas import tpu as pltpu, tpu_sc as plsc

_PI = 3.1415926


def _nodestats_body(n, bn, f_ref, p_ref, of_ref, og_ref, op_ref, os_ref):
    i = pl.program_id(0)
    f = f_ref[...]
    p = p_ref[...]
    t = f.shape[1]
    rows = i * bn + lax.broadcasted_iota(jnp.int32, (bn,), 0)
    m = rows < n
    zero = jnp.zeros((bn,), jnp.float32)
    of_ref[...] = jnp.where(m, jnp.sum(f, axis=1), zero)
    og_ref[...] = jnp.where(m, f[:, t - 1] - f[:, 0], zero)
    op_ref[...] = jnp.where(m, jnp.sum(p, axis=1), zero)
    os_ref[...] = jnp.where(m, jnp.sum(f * f, axis=1), zero)


def _node_stats(flowrate, pressure, n_pad):
    n, t = flowrate.shape
    bn = 2048
    grid = (n_pad // bn,)
    return pl.pallas_call(
        functools.partial(_nodestats_body, n, bn),
        grid=grid,
        in_specs=[
            pl.BlockSpec((bn, t), lambda i: (i, 0)),
            pl.BlockSpec((bn, t), lambda i: (i, 0)),
        ],
        out_specs=[pl.BlockSpec((bn,), lambda i: (i,))] * 4,
        out_shape=[jax.ShapeDtypeStruct((n_pad,), jnp.float32)] * 4,
    )(flowrate, pressure)


def _edge_body(n_pad, epw, ch, ft_h, gt_h, pt_h, src_h, dst_h, ln_h, dm_h,
               part_out, laste_out,
               ft_v, gt_v, pt_v, lt_v, sb, db, lb, mb, accb, sem):
    c = lax.axis_index("c")
    s = lax.axis_index("s")
    nc = lax.axis_size("c")
    wid = s * nc + c
    ebase = wid * epw
    lane = lax.iota(jnp.int32, 16)

    cps = [
        pltpu.async_copy(ft_h, ft_v, sem),
        pltpu.async_copy(gt_h, gt_v, sem),
        pltpu.async_copy(pt_h, pt_v, sem),
        pltpu.async_copy(src_h.at[pl.ds(ebase, epw)], sb, sem),
        pltpu.async_copy(dst_h.at[pl.ds(ebase, epw)], db, sem),
        pltpu.async_copy(ln_h.at[pl.ds(ebase, epw)], lb, sem),
        pltpu.async_copy(dm_h.at[pl.ds(ebase, epw)], mb, sem),
    ]

    def init_body(j, _):
        lt_v[pl.ds(j * 16, 16)] = jnp.full((16,), -1, jnp.int32)
        return 0
    lax.fori_loop(0, n_pad // 16, init_body, 0)

    for cp in cps:
        cp.wait()

    av = jnp.zeros((16,), jnp.float32)
    au = jnp.zeros((16,), jnp.float32)
    ap = jnp.zeros((16,), jnp.float32)

    for cki in range(epw // ch):
        unroll = 2

        def group_body(it, carry, _cki=cki):
            av, au, ap = carry
            for u in range(unroll):
                g = it * unroll + u
                sl = pl.ds(g * 16, 16)
                si = sb[sl]
                di = db[sl]
                ln = lb[sl]
                dm = mb[sl]
                fs = plsc.load_gather(ft_v, [si])
                fd = plsc.load_gather(ft_v, [di])
                gs = plsc.load_gather(gt_v, [si])
                gd = plsc.load_gather(gt_v, [di])
                ps = plsc.load_gather(pt_v, [si])
                pd = plsc.load_gather(pt_v, [di])
                d2 = dm * dm
                d4 = d2 * d2
                av = av + ln / d4 * (fs - fd)
                au = au + ln / d2 * (gs - gd)
                ap = ap + (ps - pd)
                e = (ebase + _cki * ch + g * 16) + lane
                plsc.store_scatter(lt_v, [di], e)
            return av, au, ap

        av, au, ap = lax.fori_loop(0, ch // 16 // unroll, group_body,
                                   (av, au, ap))

    accb[0, :] = av
    accb[1, :] = au
    accb[2, :] = ap
    accb[3, :] = jnp.zeros((16,), jnp.float32)
    pltpu.sync_copy(accb, part_out.at[wid])
    pltpu.sync_copy(lt_v, laste_out.at[wid])


def _merge_body(n, nsl, nw, laste_hbm, sq_h, dm_h, kin_out,
                l_v, ml_v, wi_v, dm_v, sq_v, ob, sem):
    c = lax.axis_index("c")
    s = lax.axis_index("s")
    nc = lax.axis_size("c")
    wid = s * nc + c
    nbase = wid * nsl
    lane = lax.iota(jnp.int32, 16)

    pltpu.sync_copy(laste_hbm.at[:, pl.ds(nbase, nsl)], l_v)
    pltpu.sync_copy(sq_h.at[pl.ds(nbase, nsl)], sq_v)

    def mbody(j, _):
        sl = pl.ds(j * 16, 16)
        ml = l_v[0, sl]
        for ti in range(1, nw):
            ml = jnp.maximum(ml, l_v[ti, sl])
        ml_v[sl] = ml
        wi_v[sl] = jnp.maximum(ml, 0)
        return 0
    lax.fori_loop(0, nsl // 16, mbody, 0)

    pick0 = (lane == 0) & (wid == 0)
    v0 = ml_v[pl.ds(0, 16)]
    ml_v[pl.ds(0, 16)] = jnp.where(pick0, v0[1], v0)
    w0 = wi_v[pl.ds(0, 16)]
    wi_v[pl.ds(0, 16)] = jnp.where(pick0, w0[1], w0)

    pltpu.async_copy(dm_h.at[wi_v], dm_v, sem).wait()

    def kbody(j, acc):
        sl = pl.ds(j * 16, 16)
        d = jnp.where(ml_v[sl] < 0, jnp.full((16,), 1.0, jnp.float32),
                      dm_v[sl])
        sq = sq_v[sl]
        d2 = d * d
        return acc + sq / (d2 * d2)
    acc = lax.fori_loop(0, nsl // 16, kbody, jnp.zeros((16,), jnp.float32))
    ob[...] = acc
    pltpu.sync_copy(ob, kin_out.at[wid])


def kernel(flowrate, pressure, edge_attr, edge_index, rho, vis, total_time):
    n, t = flowrate.shape
    e = edge_index.shape[1]

    info = plsc.get_sparse_core_info()
    nc, ns = info.num_cores, info.num_subcores
    nw = nc * ns
    n_pad = ((n + nw * 16 - 1) // (nw * 16)) * (nw * 16)
    epw = e // nw
    ch = epw
    nsl = n_pad // nw

    src = edge_index[0]
    dst = edge_index[1]
    ln = edge_attr[:, 0]
    dm = edge_attr[:, 1]

    ft, gt, pt, sq = _node_stats(flowrate, pressure, n_pad)

    mesh = plsc.VectorSubcoreMesh(core_axis_name="c", subcore_axis_name="s")
    sc_params = pltpu.CompilerParams(needs_layout_passes=False,
                                     use_tc_tiling_on_sc=False)

    edge_k = functools.partial(
        pl.kernel,
        out_type=(
            jax.ShapeDtypeStruct((nw, 4, 16), jnp.float32),
            jax.ShapeDtypeStruct((nw, n_pad), jnp.int32),
        ),
        mesh=mesh,
        scratch_types=[
            pltpu.VMEM((n_pad,), jnp.float32),
            pltpu.VMEM((n_pad,), jnp.float32),
            pltpu.VMEM((n_pad,), jnp.float32),
            pltpu.VMEM((n_pad,), jnp.int32),
            pltpu.VMEM((ch,), jnp.int32),
            pltpu.VMEM((ch,), jnp.int32),
            pltpu.VMEM((ch,), jnp.float32),
            pltpu.VMEM((ch,), jnp.float32),
            pltpu.VMEM((4, 16), jnp.float32),
            pltpu.SemaphoreType.DMA,
        ],
        compiler_params=sc_params,
    )(functools.partial(_edge_body, n_pad, epw, ch))
    part, laste_tbls = edge_k(ft, gt, pt, src, dst, ln, dm)

    merge_k = functools.partial(
        pl.kernel,
        out_type=jax.ShapeDtypeStruct((nw, 16), jnp.float32),
        mesh=mesh,
        scratch_types=[
            pltpu.VMEM((nw, nsl), jnp.int32),
            pltpu.VMEM((nsl,), jnp.int32),
            pltpu.VMEM((nsl,), jnp.int32),
            pltpu.VMEM((nsl,), jnp.float32),
            pltpu.VMEM((nsl,), jnp.float32),
            pltpu.VMEM((16,), jnp.float32),
            pltpu.SemaphoreType.DMA,
        ],
        compiler_params=sc_params,
    )(functools.partial(_merge_body, n, nsl, nw))
    kin_part = merge_k(laste_tbls, sq, dm)

    s_vis = jnp.sum(part[:, 0, :])
    s_uns = jnp.sum(part[:, 1, :])
    s_p = jnp.sum(part[:, 2, :])
    s_kin = jnp.sum(kin_part)

    rho0 = rho[0]
    loss = (16.0 * rho0 / (_PI * _PI)) * s_kin / (n * t)
    loss = loss + (128.0 * vis[0] / _PI) * s_vis / (e * t)
    loss = loss + (4.0 * rho0 / (_PI * total_time[0])) * s_uns / e
    loss = loss + s_p / (e * t)
    return loss
